# baseline (speedup 1.0000x reference)
import jax
import jax.numpy as jnp
from jax import lax
from jax.experimental import pallas as pl
from jax.experimental.pallas import tpu as pltpu
from jax.experimental.pallas import tpu_sc as plsc

_L = 200
_B = 4096
_D = 64
_V = 1000000
_BC = 256
_CHW = _B // _BC
_NU = _L * _CHW
_NC = 2
_NS = 16
_NW = _NC * _NS
_UW = _NU // _NW
_NG = _BC // 128


def _body(xt_hbm, tbl_hbm, out_hbm, ibuf, gbuf, pbuf, isem, gsem, osem):
    c = lax.axis_index("c")
    s = lax.axis_index("s")
    wid = s * _NC + c
    u_base = wid * _UW

    iota16 = lax.iota(jnp.int32, 16)

    def unit_lb(u):
        ug = u_base + u
        return ug // _CHW, lax.rem(ug, _CHW)

    def fire_idx(u, s4):
        l, bc = unit_lb(u)
        pltpu.async_copy(xt_hbm.at[l, pl.ds(bc * _NG, _NG)], ibuf.at[s4],
                         isem.at[s4])

    def wait_idx(u, s4):
        l, bc = unit_lb(u)
        pltpu.make_async_copy(xt_hbm.at[l, pl.ds(bc * _NG, _NG)],
                              ibuf.at[s4], isem.at[s4]).wait()

    def fire_gather(s4, s2):
        for k in range(_NG):
            pltpu.async_copy(tbl_hbm.at[ibuf.at[s4, k]],
                             gbuf.at[s2, pl.ds(k * 128, 128)], gsem.at[s2])

    def drain_gather(s4, s2):
        for k in range(_NG):
            pltpu.make_async_copy(tbl_hbm.at[ibuf.at[s4, k]],
                                  gbuf.at[s2, pl.ds(k * 128, 128)],
                                  gsem.at[s2]).wait()

    def fire_out(u, s2):
        l, bc = unit_lb(u)
        pltpu.async_copy(pbuf.at[s2, :, :, :, pl.ds(0, 128)],
                         out_hbm.at[l, :, pl.ds(bc * 2, 2)],
                         osem.at[s2])

    def drain_out(u, s2):
        l, bc = unit_lb(u)
        pltpu.make_async_copy(pbuf.at[s2, :, :, :, pl.ds(0, 128)],
                              out_hbm.at[l, :, pl.ds(bc * 2, 2)],
                              osem.at[s2]).wait()

    zeros16 = jnp.zeros((16,), jnp.int32)
    dgv = [(iota16 + cc * 16) >> 3 for cc in range(_D // 16)]
    dsv = [(iota16 + cc * 16) & 7 for cc in range(_D // 16)]

    def transpose_scale(s2):
        svec = zeros16 + s2

        def row(b2, c2):
            b = b2 * 4
            for r in range(4):
                btv = zeros16 + ((b + r) >> 7)
                lnv = zeros16 + ((b + r) & 127)
                for cc in range(_D // 16):
                    vals = gbuf[s2, b + r, pl.ds(cc * 16, 16)] * 8.0
                    plsc.store_scatter(pbuf, [svec, dgv[cc], btv, dsv[cc],
                                              lnv], vals)
            return c2

        lax.fori_loop(0, _BC // 4, row, 0)

    fire_idx(0, 0)
    fire_idx(1, 1)
    fire_idx(2, 2)
    wait_idx(0, 0)
    fire_gather(0, 0)

    def body(u, carry):
        s2 = lax.rem(u, 2)
        s4 = lax.rem(u, 4)
        drain_gather(s4, s2)

        @pl.when(u + 1 < _UW)
        def _():
            wait_idx(u + 1, lax.rem(u + 1, 4))
            fire_gather(lax.rem(u + 1, 4), lax.rem(u + 1, 2))

        @pl.when(u >= 2)
        def _():
            drain_out(u - 2, s2)

        transpose_scale(s2)
        fire_out(u, s2)

        @pl.when(u + 3 < _UW)
        def _():
            fire_idx(u + 3, lax.rem(u + 3, 4))

        return carry

    lax.fori_loop(0, _UW, body, 0)
    drain_out(_UW - 2, lax.rem(_UW - 2, 2))
    drain_out(_UW - 1, lax.rem(_UW - 1, 2))


def kernel(x, embedding):
    xt = (jnp.transpose(x).astype(jnp.int32) * 2).reshape(_L, _B // 128, 128)
    tbl = jnp.concatenate(
        [embedding, jnp.zeros((_V, _D), jnp.float32)], axis=1
    ).reshape(2 * _V, _D)
    mesh = plsc.VectorSubcoreMesh(core_axis_name="c", subcore_axis_name="s")
    p = pl.kernel(
        _body,
        out_type=jax.ShapeDtypeStruct((_L, _D // 8, _B // 128, 8, 128),
                                      jnp.float32),
        mesh=mesh,
        scratch_types=[
            pltpu.VMEM((4, _NG, 128), jnp.int32),
            pltpu.VMEM((2, _BC, _D), jnp.float32),
            pltpu.VMEM((2, 8, 2, 8, 129), jnp.float32),
            pltpu.SemaphoreType.DMA((4,)),
            pltpu.SemaphoreType.DMA((2,)),
            pltpu.SemaphoreType.DMA((2,)),
        ],
        compiler_params=pltpu.CompilerParams(use_tc_tiling_on_sc=False,
                                             needs_layout_passes=False),
    )(xt, tbl)
    return jnp.transpose(p, (2, 4, 0, 1, 3)).reshape(_B, _L, _D)


# speedup vs baseline: 1.4951x; 1.4951x over previous
import jax
import jax.numpy as jnp
from jax import lax
from jax.experimental import pallas as pl
from jax.experimental.pallas import tpu as pltpu
from jax.experimental.pallas import tpu_sc as plsc

_L = 200
_B = 4096
_D = 64
_V = 1000000
_BC = 256
_CHW = _B // _BC
_NU = _L * _CHW
_NC = 2
_NS = 16
_NW = _NC * _NS
_UW = _NU // _NW
_NG = _BC // 128


def _body(xt_hbm, tbl_hbm, out_hbm, ibuf, gbuf, pbuf, isem, gsem, osem):
    c = lax.axis_index("c")
    s = lax.axis_index("s")
    wid = s * _NC + c
    u_base = wid * _UW

    iota16 = lax.iota(jnp.int32, 16)

    def unit_lb(u):
        ug = u_base + u
        return ug // _CHW, lax.rem(ug, _CHW)

    def fire_idx(u, s4):
        l, bc = unit_lb(u)
        pltpu.async_copy(xt_hbm.at[l, pl.ds(bc * _NG, _NG)], ibuf.at[s4],
                         isem.at[s4])

    def wait_idx(u, s4):
        l, bc = unit_lb(u)
        pltpu.make_async_copy(xt_hbm.at[l, pl.ds(bc * _NG, _NG)],
                              ibuf.at[s4], isem.at[s4]).wait()

    def fire_gather(s4, s2):
        for k in range(_NG):
            pltpu.async_copy(tbl_hbm.at[ibuf.at[s4, k]],
                             gbuf.at[s2, pl.ds(k * 128, 128)], gsem.at[s2])

    def drain_gather(s4, s2):
        for k in range(_NG):
            pltpu.make_async_copy(tbl_hbm.at[ibuf.at[s4, k]],
                                  gbuf.at[s2, pl.ds(k * 128, 128)],
                                  gsem.at[s2]).wait()

    def fire_out(u, s2):
        l, bc = unit_lb(u)
        pltpu.async_copy(pbuf.at[s2, :, :, :, pl.ds(0, 128)],
                         out_hbm.at[l, :, pl.ds(bc * 2, 2)],
                         osem.at[s2])

    def drain_out(u, s2):
        l, bc = unit_lb(u)
        pltpu.make_async_copy(pbuf.at[s2, :, :, :, pl.ds(0, 128)],
                              out_hbm.at[l, :, pl.ds(bc * 2, 2)],
                              osem.at[s2]).wait()

    zeros16 = jnp.zeros((16,), jnp.int32)
    dgv = [(iota16 + cc * 16) >> 3 for cc in range(_D // 16)]
    dsv = [(iota16 + cc * 16) & 7 for cc in range(_D // 16)]

    def transpose_scale(s2):
        svec = zeros16 + s2

        @plsc.parallel_loop(0, _BC, 1, unroll=4)
        def _(b):
            btv = zeros16 + (b >> 7)
            lnv = zeros16 + (b & 127)
            for cc in range(_D // 16):
                vals = gbuf[s2, b, pl.ds(cc * 16, 16)] * 8.0
                plsc.store_scatter(pbuf, [svec, dgv[cc], btv, dsv[cc],
                                          lnv], vals)

    fire_idx(0, 0)
    fire_idx(1, 1)
    fire_idx(2, 2)
    wait_idx(0, 0)
    fire_gather(0, 0)

    def body(u, carry):
        s2 = lax.rem(u, 2)
        s4 = lax.rem(u, 4)
        drain_gather(s4, s2)

        @pl.when(u + 1 < _UW)
        def _():
            wait_idx(u + 1, lax.rem(u + 1, 4))
            fire_gather(lax.rem(u + 1, 4), lax.rem(u + 1, 2))

        @pl.when(u >= 2)
        def _():
            drain_out(u - 2, s2)

        transpose_scale(s2)
        fire_out(u, s2)

        @pl.when(u + 3 < _UW)
        def _():
            fire_idx(u + 3, lax.rem(u + 3, 4))

        return carry

    lax.fori_loop(0, _UW, body, 0)
    drain_out(_UW - 2, lax.rem(_UW - 2, 2))
    drain_out(_UW - 1, lax.rem(_UW - 1, 2))


def kernel(x, embedding):
    xt = (jnp.transpose(x).astype(jnp.int32) * 2).reshape(_L, _B // 128, 128)
    tbl = jnp.concatenate(
        [embedding, jnp.zeros((_V, _D), jnp.float32)], axis=1
    ).reshape(2 * _V, _D)
    mesh = plsc.VectorSubcoreMesh(core_axis_name="c", subcore_axis_name="s")
    p = pl.kernel(
        _body,
        out_type=jax.ShapeDtypeStruct((_L, _D // 8, _B // 128, 8, 128),
                                      jnp.float32),
        mesh=mesh,
        scratch_types=[
            pltpu.VMEM((4, _NG, 128), jnp.int32),
            pltpu.VMEM((2, _BC, _D), jnp.float32),
            pltpu.VMEM((2, 8, 2, 8, 129), jnp.float32),
            pltpu.SemaphoreType.DMA((4,)),
            pltpu.SemaphoreType.DMA((2,)),
            pltpu.SemaphoreType.DMA((2,)),
        ],
        compiler_params=pltpu.CompilerParams(use_tc_tiling_on_sc=False,
                                             needs_layout_passes=False),
    )(xt, tbl)
    return jnp.transpose(p, (2, 4, 0, 1, 3)).reshape(_B, _L, _D)


# speedup vs baseline: 1.4962x; 1.0007x over previous
import jax
import jax.numpy as jnp
from jax import lax
from jax.experimental import pallas as pl
from jax.experimental.pallas import tpu as pltpu
from jax.experimental.pallas import tpu_sc as plsc

_L = 200
_B = 4096
_D = 64
_V = 1000000
_BC = 256
_CHW = _B // _BC
_NU = _L * _CHW
_NC = 2
_NS = 16
_NW = _NC * _NS
_UW = _NU // _NW
_NG = _BC // 128


def _body(xt_hbm, tbl_hbm, out_hbm, ibuf, gbuf, pbuf, isem, gsem, osem):
    c = lax.axis_index("c")
    s = lax.axis_index("s")
    wid = s * _NC + c
    u_base = wid * _UW

    iota16 = lax.iota(jnp.int32, 16)

    def unit_lb(u):
        ug = u_base + u
        return ug // _CHW, lax.rem(ug, _CHW)

    def fire_idx(u, s4):
        l, bc = unit_lb(u)
        pltpu.async_copy(xt_hbm.at[l, pl.ds(bc * _NG, _NG)], ibuf.at[s4],
                         isem.at[s4])

    def wait_idx(u, s4):
        l, bc = unit_lb(u)
        pltpu.make_async_copy(xt_hbm.at[l, pl.ds(bc * _NG, _NG)],
                              ibuf.at[s4], isem.at[s4]).wait()

    def fire_gather(s4, s2):
        for k in range(_NG):
            pltpu.async_copy(tbl_hbm.at[ibuf.at[s4, k]],
                             gbuf.at[s2, pl.ds(k * 128, 128)], gsem.at[s2])

    def drain_gather(s4, s2):
        for k in range(_NG):
            pltpu.make_async_copy(tbl_hbm.at[ibuf.at[s4, k]],
                                  gbuf.at[s2, pl.ds(k * 128, 128)],
                                  gsem.at[s2]).wait()

    def fire_out(u, s2):
        l, bc = unit_lb(u)
        pltpu.async_copy(pbuf.at[s2, :, :, :, pl.ds(0, 128)],
                         out_hbm.at[l, :, pl.ds(bc * 2, 2)],
                         osem.at[s2])

    def drain_out(u, s2):
        l, bc = unit_lb(u)
        pltpu.make_async_copy(pbuf.at[s2, :, :, :, pl.ds(0, 128)],
                              out_hbm.at[l, :, pl.ds(bc * 2, 2)],
                              osem.at[s2]).wait()

    zeros16 = jnp.zeros((16,), jnp.int32)
    dgv = [(iota16 + cc * 16) >> 3 for cc in range(_D // 16)]
    dsv = [(iota16 + cc * 16) & 7 for cc in range(_D // 16)]

    def transpose_scale(s2):
        svec = zeros16 + s2

        @plsc.parallel_loop(0, _BC, 1, unroll=8)
        def _(b):
            btv = zeros16 + (b >> 7)
            lnv = zeros16 + (b & 127)
            for cc in range(_D // 16):
                vals = gbuf[s2, b, pl.ds(cc * 16, 16)] * 8.0
                plsc.store_scatter(pbuf, [svec, dgv[cc], btv, dsv[cc],
                                          lnv], vals)

    fire_idx(0, 0)
    fire_idx(1, 1)
    fire_idx(2, 2)
    wait_idx(0, 0)
    fire_gather(0, 0)

    def body(u, carry):
        s2 = lax.rem(u, 2)
        s4 = lax.rem(u, 4)
        drain_gather(s4, s2)

        @pl.when(u + 1 < _UW)
        def _():
            wait_idx(u + 1, lax.rem(u + 1, 4))
            fire_gather(lax.rem(u + 1, 4), lax.rem(u + 1, 2))

        @pl.when(u >= 2)
        def _():
            drain_out(u - 2, s2)

        transpose_scale(s2)
        fire_out(u, s2)

        @pl.when(u + 3 < _UW)
        def _():
            fire_idx(u + 3, lax.rem(u + 3, 4))

        return carry

    lax.fori_loop(0, _UW, body, 0)
    drain_out(_UW - 2, lax.rem(_UW - 2, 2))
    drain_out(_UW - 1, lax.rem(_UW - 1, 2))


def kernel(x, embedding):
    xt = (jnp.transpose(x).astype(jnp.int32) * 2).reshape(_L, _B // 128, 128)
    tbl = jnp.concatenate(
        [embedding, jnp.zeros((_V, _D), jnp.float32)], axis=1
    ).reshape(2 * _V, _D)
    mesh = plsc.VectorSubcoreMesh(core_axis_name="c", subcore_axis_name="s")
    p = pl.kernel(
        _body,
        out_type=jax.ShapeDtypeStruct((_L, _D // 8, _B // 128, 8, 128),
                                      jnp.float32),
        mesh=mesh,
        scratch_types=[
            pltpu.VMEM((4, _NG, 128), jnp.int32),
            pltpu.VMEM((2, _BC, _D), jnp.float32),
            pltpu.VMEM((2, 8, 2, 8, 129), jnp.float32),
            pltpu.SemaphoreType.DMA((4,)),
            pltpu.SemaphoreType.DMA((2,)),
            pltpu.SemaphoreType.DMA((2,)),
        ],
        compiler_params=pltpu.CompilerParams(use_tc_tiling_on_sc=False,
                                             needs_layout_passes=False),
    )(xt, tbl)
    return jnp.transpose(p, (2, 4, 0, 1, 3)).reshape(_B, _L, _D)
